# baseline (speedup 1.0000x reference)
import functools

import jax
import jax.numpy as jnp
from jax import lax
from jax.experimental import pallas as pl
from jax.experimental.pallas import tpu as pltpu
from jax.experimental.pallas import tpu_sc as plsc

NC = 2
NS = 16
NW = NC * NS
L = 16
UNROLL = 8


def _rotate(v, idx):
    dn = lax.GatherDimensionNumbers(
        offset_dims=(), collapsed_slice_dims=(0,), start_index_map=(0,))
    return lax.gather(v, idx[:, None], dn, slice_sizes=(1,),
                      mode=lax.GatherScatterMode.PROMISE_IN_BOUNDS)


def _rsqrt(v):
    magic = jnp.full((L,), 0x5F3759DF, dtype=jnp.int32)
    half = v * 0.5
    i = lax.bitcast_convert_type(v, jnp.int32)
    i = magic - lax.shift_right_logical(i, 1)
    y = lax.bitcast_convert_type(i, jnp.float32)
    for _ in range(2):
        y = y * (1.5 - half * y * y)
    return y


def kernel(inputs, token_table, pos_table, gamma, beta):
    B, S = inputs.shape
    V, H = token_table.shape
    assert H == 4 * L and B % NW == 0 and S == 200 and S % UNROLL == 0
    CA, CB = 104, 96
    SEQ_PER_W = B // NW
    PAIRS = SEQ_PER_W // 2

    pos_s = pos_table[:S]
    inputs_flat = inputs.reshape(B * S)

    mesh = plsc.VectorSubcoreMesh(core_axis_name="c", subcore_axis_name="s")

    @functools.partial(
        pl.kernel,
        out_type=jax.ShapeDtypeStruct((B, S, H), jnp.float32),
        mesh=mesh,
        compiler_params=pltpu.CompilerParams(use_tc_tiling_on_sc=False),
        scratch_types=[
            pltpu.VMEM((S, H), jnp.float32),
            pltpu.VMEM((H,), jnp.float32),
            pltpu.VMEM((H,), jnp.float32),
            pltpu.VMEM((SEQ_PER_W * S,), jnp.int32),
            pltpu.VMEM((S, H), jnp.float32),
            pltpu.VMEM((S, H), jnp.float32),
            pltpu.VMEM((S, H), jnp.float32),
            pltpu.VMEM((S, H), jnp.float32),
            pltpu.SemaphoreType.DMA,
            pltpu.SemaphoreType.DMA,
            pltpu.SemaphoreType.DMA,
            pltpu.SemaphoreType.DMA,
        ],
    )
    def k(inputs_hbm, token_hbm, pos_hbm, gamma_hbm, beta_hbm, out_hbm,
          pos_v, gamma_v, beta_v, idx_v, rows_a, rows_b, sta_a, sta_b,
          sem_ga, sem_gb, sem_oa, sem_ob):
        wid = lax.axis_index("s") * NC + lax.axis_index("c")
        base = pl.multiple_of(wid * SEQ_PER_W, 8)
        flat_base = pl.multiple_of(wid * (SEQ_PER_W * S), 8)
        pltpu.sync_copy(inputs_hbm.at[pl.ds(flat_base, SEQ_PER_W * S)], idx_v)
        pltpu.sync_copy(pos_hbm, pos_v)
        pltpu.sync_copy(gamma_hbm, gamma_v)
        pltpu.sync_copy(beta_hbm, beta_v)

        g = [gamma_v[pl.ds(i * L, L)] for i in range(4)]
        bt = [beta_v[pl.ds(i * L, L)] for i in range(4)]
        iota = lax.iota(jnp.int32, L)
        rots = [(iota + sh) & (L - 1) for sh in (1, 2, 4, 8)]

        def gather(j, rows_ref, sem):
            off = pl.multiple_of(j * S, 8)
            cp0 = pltpu.async_copy(token_hbm.at[idx_v.at[pl.ds(off, CA)]],
                                   rows_ref.at[pl.ds(0, CA)], sem)
            cp1 = pltpu.async_copy(
                token_hbm.at[idx_v.at[pl.ds(off + CA, CB)]],
                rows_ref.at[pl.ds(CA, CB)], sem)
            return cp0, cp1

        def drain_gather(j, rows_ref, sem):
            off = pl.multiple_of(j * S, 8)
            pltpu.make_async_copy(token_hbm.at[idx_v.at[pl.ds(off, CA)]],
                                  rows_ref.at[pl.ds(0, CA)], sem).wait()
            pltpu.make_async_copy(
                token_hbm.at[idx_v.at[pl.ds(off + CA, CB)]],
                rows_ref.at[pl.ds(CA, CB)], sem).wait()

        def drain_out(b, sta_ref, sem):
            pltpu.make_async_copy(sta_ref, out_hbm.at[b], sem).wait()

        def compute(rows_ref, sta_ref):
            def do_block(v, c):
                rb = v * UNROLL
                for u in range(UNROLL):
                    r = rb + u
                    x = [rows_ref[r, pl.ds(i * L, L)] +
                         pos_v[r, pl.ds(i * L, L)] for i in range(4)]
                    s = (x[0] + x[1]) + (x[2] + x[3])
                    q = ((x[0] * x[0] + x[1] * x[1]) +
                         (x[2] * x[2] + x[3] * x[3]))
                    for rot in rots:
                        s = s + _rotate(s, rot)
                        q = q + _rotate(q, rot)
                    mean = s * (1.0 / H)
                    var = q * (1.0 / H) - mean * mean
                    rstd = _rsqrt(var + 1e-12)
                    for i in range(4):
                        sta_ref[r, pl.ds(i * L, L)] = \
                            (x[i] - mean) * rstd * g[i] + bt[i]
                return c

            lax.fori_loop(0, S // UNROLL, do_block, 0)

        gather(0, rows_a, sem_ga)
        gather(1, rows_b, sem_gb)

        def pair(t, c):
            for (par, rows_ref, sta_ref, sem_g, sem_o) in (
                    (0, rows_a, sta_a, sem_ga, sem_oa),
                    (1, rows_b, sta_b, sem_gb, sem_ob)):
                j = 2 * t + par
                b = base + j
                drain_gather(j, rows_ref, sem_g)

                @pl.when(t > 0)
                def _():
                    drain_out(b, sta_ref, sem_o)

                compute(rows_ref, sta_ref)
                pltpu.async_copy(sta_ref, out_hbm.at[b], sem_o)

                @pl.when(t < PAIRS - 1)
                def _():
                    gather(j + 2, rows_ref, sem_g)
            return c

        lax.fori_loop(0, PAIRS, pair, 0)
        drain_out(base, sta_a, sem_oa)
        drain_out(base, sta_b, sem_ob)

    return k(inputs_flat, token_table, pos_s, gamma, beta)


# speedup vs baseline: 1.0303x; 1.0303x over previous
import functools

import jax
import jax.numpy as jnp
from jax import lax
from jax.experimental import pallas as pl
from jax.experimental.pallas import tpu as pltpu
from jax.experimental.pallas import tpu_sc as plsc

NC = 2
NS = 16
NW = NC * NS
L = 16
UNROLL = 5


def _rotate(v, idx):
    dn = lax.GatherDimensionNumbers(
        offset_dims=(), collapsed_slice_dims=(0,), start_index_map=(0,))
    return lax.gather(v, idx[:, None], dn, slice_sizes=(1,),
                      mode=lax.GatherScatterMode.PROMISE_IN_BOUNDS)


def _rsqrt(v):
    magic = jnp.full((L,), 0x5F3759DF, dtype=jnp.int32)
    half = v * 0.5
    i = lax.bitcast_convert_type(v, jnp.int32)
    i = magic - lax.shift_right_logical(i, 1)
    y = lax.bitcast_convert_type(i, jnp.float32)
    for _ in range(2):
        y = y * (1.5 - half * y * y)
    return y


def kernel(inputs, token_table, pos_table, gamma, beta):
    B, S = inputs.shape
    V, H = token_table.shape
    assert H == 4 * L and B % NW == 0 and S == 200 and S % UNROLL == 0
    CA, CB = 104, 96
    SEQ_PER_W = B // NW
    PAIRS = SEQ_PER_W // 2

    pos_s = pos_table[:S]
    inputs_flat = inputs.reshape(B * S)

    mesh = plsc.VectorSubcoreMesh(core_axis_name="c", subcore_axis_name="s")

    @functools.partial(
        pl.kernel,
        out_type=jax.ShapeDtypeStruct((B, S, H), jnp.float32),
        mesh=mesh,
        compiler_params=pltpu.CompilerParams(use_tc_tiling_on_sc=False),
        scratch_types=[
            pltpu.VMEM((S, H), jnp.float32),
            pltpu.VMEM((H,), jnp.float32),
            pltpu.VMEM((H,), jnp.float32),
            pltpu.VMEM((SEQ_PER_W * S,), jnp.int32),
            pltpu.VMEM((S, H), jnp.float32),
            pltpu.VMEM((S, H), jnp.float32),
            pltpu.VMEM((S, H), jnp.float32),
            pltpu.VMEM((S, H), jnp.float32),
            pltpu.SemaphoreType.DMA,
            pltpu.SemaphoreType.DMA,
            pltpu.SemaphoreType.DMA,
            pltpu.SemaphoreType.DMA,
        ],
    )
    def k(inputs_hbm, token_hbm, pos_hbm, gamma_hbm, beta_hbm, out_hbm,
          pos_v, gamma_v, beta_v, idx_v, rows_a, rows_b, sta_a, sta_b,
          sem_ga, sem_gb, sem_oa, sem_ob):
        wid = lax.axis_index("s") * NC + lax.axis_index("c")
        base = pl.multiple_of(wid * SEQ_PER_W, 8)
        flat_base = pl.multiple_of(wid * (SEQ_PER_W * S), 8)
        pltpu.sync_copy(inputs_hbm.at[pl.ds(flat_base, SEQ_PER_W * S)], idx_v)
        pltpu.sync_copy(pos_hbm, pos_v)
        pltpu.sync_copy(gamma_hbm, gamma_v)
        pltpu.sync_copy(beta_hbm, beta_v)

        g = [gamma_v[pl.ds(i * L, L)] for i in range(4)]
        bt = [beta_v[pl.ds(i * L, L)] for i in range(4)]
        iota = lax.iota(jnp.int32, L)
        rots = [(iota + sh) & (L - 1) for sh in (1, 2, 4, 8)]

        def gather(j, rows_ref, sem):
            off = pl.multiple_of(j * S, 8)
            cp0 = pltpu.async_copy(token_hbm.at[idx_v.at[pl.ds(off, CA)]],
                                   rows_ref.at[pl.ds(0, CA)], sem)
            cp1 = pltpu.async_copy(
                token_hbm.at[idx_v.at[pl.ds(off + CA, CB)]],
                rows_ref.at[pl.ds(CA, CB)], sem)
            return cp0, cp1

        def drain_gather(j, rows_ref, sem):
            off = pl.multiple_of(j * S, 8)
            pltpu.make_async_copy(token_hbm.at[idx_v.at[pl.ds(off, CA)]],
                                  rows_ref.at[pl.ds(0, CA)], sem).wait()
            pltpu.make_async_copy(
                token_hbm.at[idx_v.at[pl.ds(off + CA, CB)]],
                rows_ref.at[pl.ds(CA, CB)], sem).wait()

        def drain_out(b, sta_ref, sem):
            pltpu.make_async_copy(sta_ref, out_hbm.at[b], sem).wait()

        def compute(rows_ref, sta_ref):
            def do_block(v, c):
                rb = v * UNROLL
                for u in range(UNROLL):
                    r = rb + u
                    x = [rows_ref[r, pl.ds(i * L, L)] +
                         pos_v[r, pl.ds(i * L, L)] for i in range(4)]
                    s = (x[0] + x[1]) + (x[2] + x[3])
                    q = ((x[0] * x[0] + x[1] * x[1]) +
                         (x[2] * x[2] + x[3] * x[3]))
                    for rot in rots:
                        s = s + _rotate(s, rot)
                        q = q + _rotate(q, rot)
                    mean = s * (1.0 / H)
                    var = q * (1.0 / H) - mean * mean
                    rstd = _rsqrt(var + 1e-12)
                    for i in range(4):
                        sta_ref[r, pl.ds(i * L, L)] = \
                            (x[i] - mean) * rstd * g[i] + bt[i]
                return c

            lax.fori_loop(0, S // UNROLL, do_block, 0)

        gather(0, rows_a, sem_ga)
        gather(1, rows_b, sem_gb)

        def pair(t, c):
            for (par, rows_ref, sta_ref, sem_g, sem_o) in (
                    (0, rows_a, sta_a, sem_ga, sem_oa),
                    (1, rows_b, sta_b, sem_gb, sem_ob)):
                j = 2 * t + par
                b = base + j
                drain_gather(j, rows_ref, sem_g)

                @pl.when(t > 0)
                def _():
                    drain_out(b, sta_ref, sem_o)

                compute(rows_ref, sta_ref)
                pltpu.async_copy(sta_ref, out_hbm.at[b], sem_o)

                @pl.when(t < PAIRS - 1)
                def _():
                    gather(j + 2, rows_ref, sem_g)
            return c

        lax.fori_loop(0, PAIRS, pair, 0)
        drain_out(base, sta_a, sem_oa)
        drain_out(base, sta_b, sem_ob)

    return k(inputs_flat, token_table, pos_s, gamma, beta)
